# initial kernel scaffold (unmeasured)
import jax
import jax.numpy as jnp
from jax import lax
from jax.experimental import pallas as pl
from jax.experimental.pallas import tpu as pltpu


def kernel(
    x,
):
    def body(*refs):
        pass

    out_shape = jax.ShapeDtypeStruct(..., jnp.float32)
    return pl.pallas_call(body, out_shape=out_shape)(...)


# baseline (device time: 29798 ns/iter reference)
import jax
import jax.numpy as jnp
from jax import lax
from jax.experimental import pallas as pl
from jax.experimental.pallas import tpu as pltpu

N_COLS = 512


def kernel(x):
    _, m, n = x.shape

    def body(x_ref, out_ref, comm_ref, send_sem, recv_sem):
        my_x = lax.axis_index("x")
        my_y = lax.axis_index("y")
        peer = (my_x, 1 - my_y)

        barrier_sem = pltpu.get_barrier_semaphore()
        pl.semaphore_signal(
            barrier_sem, inc=1, device_id=peer,
            device_id_type=pl.DeviceIdType.MESH,
        )
        pl.semaphore_wait(barrier_sem, 1)

        rdma = pltpu.make_async_remote_copy(
            src_ref=x_ref.at[0, :, pl.ds((1 - my_y) * N_COLS, N_COLS)],
            dst_ref=comm_ref,
            send_sem=send_sem,
            recv_sem=recv_sem,
            device_id=peer,
            device_id_type=pl.DeviceIdType.MESH,
        )
        rdma.start()
        rdma.wait()

        out_ref[:, :] = (
            x_ref[0, :, pl.ds(my_y * N_COLS, N_COLS)] + comm_ref[:, :]
        )

    return pl.pallas_call(
        body,
        out_shape=jax.ShapeDtypeStruct((m, N_COLS), jnp.float32),
        in_specs=[pl.BlockSpec(memory_space=pltpu.VMEM)],
        out_specs=pl.BlockSpec(memory_space=pltpu.VMEM),
        scratch_shapes=[
            pltpu.VMEM((m, N_COLS), jnp.float32),
            pltpu.SemaphoreType.DMA,
            pltpu.SemaphoreType.DMA,
        ],
        compiler_params=pltpu.CompilerParams(collective_id=0),
    )(x)


# device time: 23476 ns/iter; 1.2693x vs baseline; 1.2693x over previous
import jax
import jax.numpy as jnp
from jax import lax
from jax.experimental import pallas as pl
from jax.experimental.pallas import tpu as pltpu

N_COLS = 512
HALF_M = 512
N_CHUNK = 4
CSZ = HALF_M // N_CHUNK


def kernel(x):
    _, m, n = x.shape

    def body(x_ref, out_ref, ybuf, y_send, y_recv, x_send, x_recv):
        my_x = lax.axis_index("x")
        my_y = lax.axis_index("y")
        y_peer = (my_x, 1 - my_y)
        x_peer = (1 - my_x, my_y)

        barrier_sem = pltpu.get_barrier_semaphore()
        for nbr in (y_peer, x_peer):
            pl.semaphore_signal(
                barrier_sem, inc=1, device_id=nbr,
                device_id_type=pl.DeviceIdType.MESH,
            )
        pl.semaphore_wait(barrier_sem, 2)

        row0 = my_x * HALF_M
        peer_row0 = (1 - my_x) * HALF_M

        y_rdmas = []
        for c in range(N_CHUNK):
            rdma = pltpu.make_async_remote_copy(
                src_ref=x_ref.at[
                    0,
                    pl.ds(row0 + c * CSZ, CSZ),
                    pl.ds((1 - my_y) * N_COLS, N_COLS),
                ],
                dst_ref=ybuf.at[pl.ds(c * CSZ, CSZ), :],
                send_sem=y_send.at[c],
                recv_sem=y_recv.at[c],
                device_id=y_peer,
                device_id_type=pl.DeviceIdType.MESH,
            )
            rdma.start()
            y_rdmas.append(rdma)

        x_rdmas = []
        for c in range(N_CHUNK):
            y_rdmas[c].wait_recv()
            rows = pl.ds(row0 + c * CSZ, CSZ)
            out_ref[rows, :] = (
                x_ref[0, rows, pl.ds(my_y * N_COLS, N_COLS)]
                + ybuf[pl.ds(c * CSZ, CSZ), :]
            )
            rdma = pltpu.make_async_remote_copy(
                src_ref=out_ref.at[rows, :],
                dst_ref=out_ref.at[rows, :],
                send_sem=x_send.at[c],
                recv_sem=x_recv.at[c],
                device_id=x_peer,
                device_id_type=pl.DeviceIdType.MESH,
            )
            rdma.start()
            x_rdmas.append(rdma)

        for c in range(N_CHUNK):
            recv = pltpu.make_async_remote_copy(
                src_ref=out_ref.at[pl.ds(peer_row0 + c * CSZ, CSZ), :],
                dst_ref=out_ref.at[pl.ds(peer_row0 + c * CSZ, CSZ), :],
                send_sem=x_send.at[c],
                recv_sem=x_recv.at[c],
                device_id=x_peer,
                device_id_type=pl.DeviceIdType.MESH,
            )
            recv.wait_recv()

        for c in range(N_CHUNK):
            y_rdmas[c].wait_send()
            x_rdmas[c].wait_send()

    return pl.pallas_call(
        body,
        out_shape=jax.ShapeDtypeStruct((m, N_COLS), jnp.float32),
        in_specs=[pl.BlockSpec(memory_space=pltpu.VMEM)],
        out_specs=pl.BlockSpec(memory_space=pltpu.VMEM),
        scratch_shapes=[
            pltpu.VMEM((HALF_M, N_COLS), jnp.float32),
            pltpu.SemaphoreType.DMA((N_CHUNK,)),
            pltpu.SemaphoreType.DMA((N_CHUNK,)),
            pltpu.SemaphoreType.DMA((N_CHUNK,)),
            pltpu.SemaphoreType.DMA((N_CHUNK,)),
        ],
        compiler_params=pltpu.CompilerParams(collective_id=0),
    )(x)


# device time: 22680 ns/iter; 1.3138x vs baseline; 1.0351x over previous
import jax
import jax.numpy as jnp
from jax import lax
from jax.experimental import pallas as pl
from jax.experimental.pallas import tpu as pltpu

N_COLS = 512
HALF_M = 512
N_CHUNK = 8
CSZ = HALF_M // N_CHUNK


def kernel(x):
    _, m, n = x.shape

    def body(
        x_hbm, out_ref, ybuf, xloc, y_send, y_recv, x_send, x_recv, loc_sem
    ):
        my_x = lax.axis_index("x")
        my_y = lax.axis_index("y")
        y_peer = (my_x, 1 - my_y)
        x_peer = (1 - my_x, my_y)

        row0 = my_x * HALF_M
        peer_row0 = (1 - my_x) * HALF_M

        loc = pltpu.make_async_copy(
            x_hbm.at[0, pl.ds(row0, HALF_M), pl.ds(my_y * N_COLS, N_COLS)],
            xloc,
            loc_sem,
        )
        loc.start()

        barrier_sem = pltpu.get_barrier_semaphore()
        for nbr in (y_peer, x_peer):
            pl.semaphore_signal(
                barrier_sem, inc=1, device_id=nbr,
                device_id_type=pl.DeviceIdType.MESH,
            )
        pl.semaphore_wait(barrier_sem, 2)

        y_rdmas = []
        for c in range(N_CHUNK):
            rdma = pltpu.make_async_remote_copy(
                src_ref=x_hbm.at[
                    0,
                    pl.ds(row0 + c * CSZ, CSZ),
                    pl.ds((1 - my_y) * N_COLS, N_COLS),
                ],
                dst_ref=ybuf.at[pl.ds(c * CSZ, CSZ), :],
                send_sem=y_send.at[c],
                recv_sem=y_recv.at[c],
                device_id=y_peer,
                device_id_type=pl.DeviceIdType.MESH,
            )
            rdma.start()
            y_rdmas.append(rdma)

        loc.wait()

        x_rdmas = []
        for c in range(N_CHUNK):
            y_rdmas[c].wait_recv()
            cs = pl.ds(c * CSZ, CSZ)
            rows = pl.ds(row0 + c * CSZ, CSZ)
            out_ref[rows, :] = xloc[cs, :] + ybuf[cs, :]
            rdma = pltpu.make_async_remote_copy(
                src_ref=out_ref.at[rows, :],
                dst_ref=out_ref.at[rows, :],
                send_sem=x_send.at[c],
                recv_sem=x_recv.at[c],
                device_id=x_peer,
                device_id_type=pl.DeviceIdType.MESH,
            )
            rdma.start()
            x_rdmas.append(rdma)

        for c in range(N_CHUNK):
            recv = pltpu.make_async_remote_copy(
                src_ref=out_ref.at[pl.ds(peer_row0 + c * CSZ, CSZ), :],
                dst_ref=out_ref.at[pl.ds(peer_row0 + c * CSZ, CSZ), :],
                send_sem=x_send.at[c],
                recv_sem=x_recv.at[c],
                device_id=x_peer,
                device_id_type=pl.DeviceIdType.MESH,
            )
            recv.wait_recv()

        for c in range(N_CHUNK):
            y_rdmas[c].wait_send()
            x_rdmas[c].wait_send()

    return pl.pallas_call(
        body,
        out_shape=jax.ShapeDtypeStruct((m, N_COLS), jnp.float32),
        in_specs=[pl.BlockSpec(memory_space=pl.ANY)],
        out_specs=pl.BlockSpec(memory_space=pltpu.VMEM),
        scratch_shapes=[
            pltpu.VMEM((HALF_M, N_COLS), jnp.float32),
            pltpu.VMEM((HALF_M, N_COLS), jnp.float32),
            pltpu.SemaphoreType.DMA((N_CHUNK,)),
            pltpu.SemaphoreType.DMA((N_CHUNK,)),
            pltpu.SemaphoreType.DMA((N_CHUNK,)),
            pltpu.SemaphoreType.DMA((N_CHUNK,)),
            pltpu.SemaphoreType.DMA,
        ],
        compiler_params=pltpu.CompilerParams(collective_id=0),
    )(x)


# device time: 19883 ns/iter; 1.4987x vs baseline; 1.1407x over previous
import os

import jax
import jax.numpy as jnp
from jax import lax
from jax.experimental import pallas as pl
from jax.experimental.pallas import tpu as pltpu

_PROBE = os.environ.get("PROBE", "")

N_COLS = 512
HALF_M = 512
N_CHUNK = int(os.environ.get("N_CHUNK", "8"))
CSZ = HALF_M // N_CHUNK


def kernel(x):
    _, m, n = x.shape

    def body(
        x_hbm, out_ref, ybuf, xloc, y_send, y_recv, x_send, x_recv, loc_sem
    ):
        my_x = lax.axis_index("x")
        my_y = lax.axis_index("y")
        y_peer = (my_x, 1 - my_y)
        x_peer = (1 - my_x, my_y)

        row0 = my_x * HALF_M
        peer_row0 = (1 - my_x) * HALF_M

        loc = pltpu.make_async_copy(
            x_hbm.at[0, pl.ds(row0, HALF_M), pl.ds(my_y * N_COLS, N_COLS)],
            xloc,
            loc_sem,
        )
        loc.start()

        barrier_sem = pltpu.get_barrier_semaphore()
        for nbr in (y_peer, x_peer):
            pl.semaphore_signal(
                barrier_sem, inc=1, device_id=nbr,
                device_id_type=pl.DeviceIdType.MESH,
            )
        pl.semaphore_wait(barrier_sem, 2)

        y_rdmas = []
        if _PROBE != "x":
            for c in range(N_CHUNK):
                rdma = pltpu.make_async_remote_copy(
                    src_ref=x_hbm.at[
                        0,
                        pl.ds(row0 + c * CSZ, CSZ),
                        pl.ds((1 - my_y) * N_COLS, N_COLS),
                    ],
                    dst_ref=ybuf.at[pl.ds(c * CSZ, CSZ), :],
                    send_sem=y_send.at[c],
                    recv_sem=y_recv.at[c],
                    device_id=y_peer,
                    device_id_type=pl.DeviceIdType.MESH,
                )
                rdma.start()
                y_rdmas.append(rdma)

        loc.wait()

        x_rdmas = []
        for c in range(N_CHUNK):
            if _PROBE != "x":
                y_rdmas[c].wait_recv()
            cs = pl.ds(c * CSZ, CSZ)
            rows = pl.ds(row0 + c * CSZ, CSZ)
            out_ref[rows, :] = xloc[cs, :] + ybuf[cs, :]
            if _PROBE != "y":
                rdma = pltpu.make_async_remote_copy(
                    src_ref=out_ref.at[rows, :],
                    dst_ref=out_ref.at[rows, :],
                    send_sem=x_send.at[c],
                    recv_sem=x_recv.at[c],
                    device_id=x_peer,
                    device_id_type=pl.DeviceIdType.MESH,
                )
                rdma.start()
                x_rdmas.append(rdma)

        if _PROBE != "y":
            for c in range(N_CHUNK):
                recv = pltpu.make_async_remote_copy(
                    src_ref=out_ref.at[pl.ds(peer_row0 + c * CSZ, CSZ), :],
                    dst_ref=out_ref.at[pl.ds(peer_row0 + c * CSZ, CSZ), :],
                    send_sem=x_send.at[c],
                    recv_sem=x_recv.at[c],
                    device_id=x_peer,
                    device_id_type=pl.DeviceIdType.MESH,
                )
                recv.wait_recv()

        for rdma in y_rdmas + x_rdmas:
            rdma.wait_send()

    return pl.pallas_call(
        body,
        out_shape=jax.ShapeDtypeStruct((m, N_COLS), jnp.float32),
        in_specs=[pl.BlockSpec(memory_space=pl.ANY)],
        out_specs=pl.BlockSpec(memory_space=pltpu.VMEM),
        scratch_shapes=[
            pltpu.VMEM((HALF_M, N_COLS), jnp.float32),
            pltpu.VMEM((HALF_M, N_COLS), jnp.float32),
            pltpu.SemaphoreType.DMA((N_CHUNK,)),
            pltpu.SemaphoreType.DMA((N_CHUNK,)),
            pltpu.SemaphoreType.DMA((N_CHUNK,)),
            pltpu.SemaphoreType.DMA((N_CHUNK,)),
            pltpu.SemaphoreType.DMA,
        ],
        compiler_params=pltpu.CompilerParams(collective_id=0),
    )(x)
